# baseline (speedup 1.0000x reference)
import functools

import jax
import jax.numpy as jnp
from jax.experimental import pallas as pl

NUM_FREQ = 12
EMB_DIM = 22
N_VARS = 100
CH = 2 * NUM_FREQ + EMB_DIM


def _geo_kernel(aux_ref, tp_ref, freq_ref, off_ref, out_ref, mask_ref):
    aux = aux_ref[...]
    ang = aux[:, :, None] * freq_ref[0][None, None, :] + off_ref[0][None, None, :]
    trig = jnp.sin(ang)
    ch = jax.lax.broadcasted_iota(jnp.int32, trig.shape, 2)
    tp = tp_ref[...][None, :, :]
    out_ref[...] = jnp.where(ch < 2 * NUM_FREQ, trig, tp)
    m = jnp.isnan(aux)
    mask_ref[...] = jnp.broadcast_to(m[:, None, :], mask_ref.shape)


@functools.partial(jax.jit, static_argnames=())
def kernel(aux_values, predictor_values, table):
    B, V = aux_values.shape
    L = predictor_values.shape[1]
    freqs = jnp.pi * (2.0 ** jnp.arange(NUM_FREQ, dtype=jnp.float32))
    freq_vec = jnp.concatenate(
        [freqs, freqs, jnp.zeros((EMB_DIM,), jnp.float32)])[None, :]
    off_vec = jnp.concatenate(
        [jnp.zeros((NUM_FREQ,), jnp.float32),
         jnp.full((NUM_FREQ,), jnp.pi / 2, jnp.float32),
         jnp.zeros((EMB_DIM,), jnp.float32)])[None, :]
    tablepad = jnp.concatenate(
        [jnp.zeros((V, 2 * NUM_FREQ), jnp.float32), table], axis=1)

    Bb = 128
    grid = (B // Bb,)
    out, mask = pl.pallas_call(
        _geo_kernel,
        grid=grid,
        in_specs=[
            pl.BlockSpec((Bb, V), lambda i: (i, 0)),
            pl.BlockSpec((V, CH), lambda i: (0, 0)),
            pl.BlockSpec((1, CH), lambda i: (0, 0)),
            pl.BlockSpec((1, CH), lambda i: (0, 0)),
        ],
        out_specs=[
            pl.BlockSpec((Bb, V, CH), lambda i: (i, 0, 0)),
            pl.BlockSpec((Bb, L, V), lambda i: (i, 0, 0)),
        ],
        out_shape=[
            jax.ShapeDtypeStruct((B, V, CH), jnp.float32),
            jax.ShapeDtypeStruct((B, L, V), jnp.bool_),
        ],
    )(aux_values, tablepad, freq_vec, off_vec)
    return (out, mask)


# speedup vs baseline: 1.9383x; 1.9383x over previous
import functools

import jax
import jax.numpy as jnp
from jax.experimental import pallas as pl

NUM_FREQ = 12
EMB_DIM = 22
N_VARS = 100
CH = 2 * NUM_FREQ + EMB_DIM


_C0 = 0.9999999724233232
_C1 = -4.934801712206059
_C2 = 4.058698080034174
_C3 = -1.3349074889428313
_C4 = 0.2314170649460755


def _sinpi(t):
    ft = jnp.floor(t)
    u = t - ft
    half = ft * 0.5
    par = half - jnp.floor(half)
    sign = 1.0 - 4.0 * par
    w = u - 0.5
    z = w * w
    p = _C4
    p = p * z + _C3
    p = p * z + _C2
    p = p * z + _C1
    p = p * z + _C0
    return p * sign


def _geo_kernel(aux_ref, tp_ref, freq_ref, off_ref, out_ref, mask_ref):
    aux = aux_ref[...]
    t = aux[:, :, None] * freq_ref[0][None, None, :] + off_ref[0][None, None, :]
    trig = _sinpi(t)
    ch = jax.lax.broadcasted_iota(jnp.int32, trig.shape, 2)
    tp = tp_ref[...][None, :, :]
    out_ref[...] = jnp.where(ch < 2 * NUM_FREQ, trig, tp)
    m = jnp.isnan(aux)
    mask_ref[...] = jnp.broadcast_to(m[:, None, :], mask_ref.shape)


@functools.partial(jax.jit, static_argnames=())
def kernel(aux_values, predictor_values, table):
    B, V = aux_values.shape
    L = predictor_values.shape[1]
    freqs = 2.0 ** jnp.arange(NUM_FREQ, dtype=jnp.float32)
    freq_vec = jnp.concatenate(
        [freqs, freqs, jnp.zeros((EMB_DIM,), jnp.float32)])[None, :]
    off_vec = jnp.concatenate(
        [jnp.zeros((NUM_FREQ,), jnp.float32),
         jnp.full((NUM_FREQ,), 0.5, jnp.float32),
         jnp.zeros((EMB_DIM,), jnp.float32)])[None, :]
    tablepad = jnp.concatenate(
        [jnp.zeros((V, 2 * NUM_FREQ), jnp.float32), table], axis=1)

    Bb = 128
    grid = (B // Bb,)
    out, mask = pl.pallas_call(
        _geo_kernel,
        grid=grid,
        in_specs=[
            pl.BlockSpec((Bb, V), lambda i: (i, 0)),
            pl.BlockSpec((V, CH), lambda i: (0, 0)),
            pl.BlockSpec((1, CH), lambda i: (0, 0)),
            pl.BlockSpec((1, CH), lambda i: (0, 0)),
        ],
        out_specs=[
            pl.BlockSpec((Bb, V, CH), lambda i: (i, 0, 0)),
            pl.BlockSpec((Bb, L, V), lambda i: (i, 0, 0)),
        ],
        out_shape=[
            jax.ShapeDtypeStruct((B, V, CH), jnp.float32),
            jax.ShapeDtypeStruct((B, L, V), jnp.bool_),
        ],
    )(aux_values, tablepad, freq_vec, off_vec)
    return (out, mask)


# speedup vs baseline: 19.7831x; 10.2063x over previous
import functools

import jax
import jax.numpy as jnp
from jax.experimental import pallas as pl

NUM_FREQ = 12
EMB_DIM = 22
N_VARS = 100
CH = 2 * NUM_FREQ + EMB_DIM

_C = (0.9999999995124089, -4.934802118487793, 4.05870883800603,
      -1.3352100152568833, 0.23493326541101656, -0.02439611339077682)


def _sinpi(t):
    ft = jnp.floor(t)
    u = t - ft
    half = ft * 0.5
    par = half - jnp.floor(half)
    sign = 1.0 - 4.0 * par
    w = u - 0.5
    z = w * w
    p = _C[5]
    p = p * z + _C[4]
    p = p * z + _C[3]
    p = p * z + _C[2]
    p = p * z + _C[1]
    p = p * z + _C[0]
    return p * sign


def _geo_kernel(auxT_ref, tbl_ref, out_ref, mask_ref):
    x = auxT_ref[...]
    s = _sinpi(x)
    c = _sinpi(x + 0.5)
    out_ref[0] = s
    out_ref[NUM_FREQ] = c
    for k in range(1, NUM_FREQ):
        s, c = 2.0 * s * c, 1.0 - 2.0 * s * s
        out_ref[k] = s
        out_ref[NUM_FREQ + k] = c
    e = tbl_ref[...]
    out_ref[2 * NUM_FREQ:] = jnp.broadcast_to(e, (EMB_DIM,) + x.shape)
    mask_ref[...] = (x != x).astype(jnp.int8)


@functools.partial(jax.jit, static_argnames=())
def kernel(aux_values, predictor_values, table):
    B, V = aux_values.shape
    L = predictor_values.shape[1]
    auxT = aux_values.T
    tblT = table.T.reshape(EMB_DIM, V, 1)

    Bb = 256
    grid = (B // Bb,)
    outT, maskT = pl.pallas_call(
        _geo_kernel,
        grid=grid,
        in_specs=[
            pl.BlockSpec((V, Bb), lambda i: (0, i)),
            pl.BlockSpec((EMB_DIM, V, 1), lambda i: (0, 0, 0)),
        ],
        out_specs=[
            pl.BlockSpec((CH, V, Bb), lambda i: (0, 0, i)),
            pl.BlockSpec((V, Bb), lambda i: (0, i)),
        ],
        out_shape=[
            jax.ShapeDtypeStruct((CH, V, B), jnp.float32),
            jax.ShapeDtypeStruct((V, B), jnp.int8),
        ],
    )(auxT, tblT)
    out = outT.transpose(2, 1, 0)
    mask = jnp.broadcast_to((maskT.T != 0)[:, None, :], (B, L, V))
    return (out, mask)
